# Initial kernel scaffold; baseline (speedup 1.0000x reference)
import jax
import jax.numpy as jnp
from jax.experimental import pallas as pl


def kernel(x, edge_index, com_xs, pos_emb, lap_pe, Wp, bp, W1, b1, W2, b2, W3, b3, Wc1, bc1, Wc2, bc2, Wc3, bc3):
    raise NotImplementedError("write your pallas kernel here")


# speedup vs baseline: 2.4322x; 2.4322x over previous
import functools

import jax
import jax.numpy as jnp
from jax import lax
from jax.experimental import pallas as pl
from jax.experimental.pallas import tpu as pltpu

_N = 10000
_NPAD = 10240
_F = 128
_R = 1024


def _dotT(a, w):
    return lax.dot_general(a, w, (((1,), (1,)), ((), ())),
                           preferred_element_type=jnp.float32)


def _dot(a, w):
    return lax.dot_general(a, w, (((1,), (0,)), ((), ())),
                           preferred_element_type=jnp.float32)


def _lrelu(v):
    return jnp.where(v >= 0, v, 0.01 * v)


def _prologue_body(x_ref, pe_ref, Wp_ref, bp_ref, deg_ref,
                   x0_ref, dinv_ref):
    pos = _dotT(pe_ref[...], Wp_ref[...]) + bp_ref[...]
    x0_ref[...] = jnp.concatenate([x_ref[...], pos], axis=1)
    dinv_ref[...] = lax.rsqrt(deg_ref[...] + 1.0)


def _community_body(x0_ref, m_ref, W1_ref, b1_ref, W2_ref, b2_ref,
                    W3_ref, b3_ref, Wc1_ref, dinv_ref, y_ref):
    x = x0_ref[...]
    for i in range(4):
        t = _lrelu(_dotT(x, W1_ref[i]) + b1_ref[i:i + 1, :])
        t = _lrelu(_dotT(t, W2_ref[i]) + b2_ref[i:i + 1, :])
        t = _lrelu(_dotT(t, W3_ref[i]) + b3_ref[i:i + 1, :])
        x = jnp.where(m_ref[:, i:i + 1] > 0, t, x)
    y_ref[...] = _dot(x, Wc1_ref[...]) * dinv_ref[...]


def _mid_body(acc_ref, y_ref, dinv_ref, b_ref, Wn_ref, y2_ref):
    z = jnp.maximum(dinv_ref[...] * (acc_ref[...] + y_ref[...]) + b_ref[...],
                    0.0)
    y2_ref[...] = _dot(z, Wn_ref[...]) * dinv_ref[...]


def _final_body(acc_ref, y_ref, dinv_ref, b_ref, out_ref):
    out_ref[...] = dinv_ref[...] * (acc_ref[...] + y_ref[...]) + b_ref[...]


def _rows(shape):
    return pl.BlockSpec(shape, lambda i: (i,) + (0,) * (len(shape) - 1))


def _full(shape):
    return pl.BlockSpec(shape, lambda i: (0,) * len(shape))


def _tc_prologue(xp, pe, Wp, bp, deg):
    g = _NPAD // _R
    return pl.pallas_call(
        _prologue_body,
        grid=(g,),
        in_specs=[_rows((_R, _F - 4)), _rows((_R, 15)), _full((4, 15)),
                  _full((1, 4)), _rows((_R, 1))],
        out_specs=[_rows((_R, _F)), _rows((_R, 1))],
        out_shape=[jax.ShapeDtypeStruct((_NPAD, _F), jnp.float32),
                   jax.ShapeDtypeStruct((_NPAD, 1), jnp.float32)],
    )(xp, pe, Wp, bp, deg)


def _tc_community(x0, masks, W1, b1, W2, b2, W3, b3, Wc1, dinv):
    g = _NPAD // _R
    return pl.pallas_call(
        _community_body,
        grid=(g,),
        in_specs=[_rows((_R, _F)), _rows((_R, 4)),
                  _full((4, _F, _F)), _full((4, _F)),
                  _full((4, _F, _F)), _full((4, _F)),
                  _full((4, _F, _F)), _full((4, _F)),
                  _full((_F, _F)), _rows((_R, 1))],
        out_specs=_rows((_R, _F)),
        out_shape=jax.ShapeDtypeStruct((_NPAD, _F), jnp.float32),
    )(x0, masks, W1, b1, W2, b2, W3, b3, Wc1, dinv)


def _tc_mid(acc, y, dinv, b, Wn):
    g = _NPAD // _R
    return pl.pallas_call(
        _mid_body,
        grid=(g,),
        in_specs=[_rows((_R, _F)), _rows((_R, _F)), _rows((_R, 1)),
                  _full((1, _F)), _full((_F, _F))],
        out_specs=_rows((_R, _F)),
        out_shape=jax.ShapeDtypeStruct((_NPAD, _F), jnp.float32),
    )(acc, y, dinv, b, Wn)


def _tc_final(acc, y, dinv, b):
    g = _NPAD // _R
    return pl.pallas_call(
        _final_body,
        grid=(g,),
        in_specs=[_rows((_R, _F)), _rows((_R, _F)), _rows((_R, 1)),
                  _full((1, _F))],
        out_specs=_rows((_R, _F)),
        out_shape=jax.ShapeDtypeStruct((_NPAD, _F), jnp.float32),
    )(acc, y, dinv, b)


def _edge_acc(y, row, col):
    return jax.ops.segment_sum(y[row], col, num_segments=_NPAD)


def kernel(x, edge_index, com_xs, pos_emb, lap_pe, Wp, bp,
           W1, b1, W2, b2, W3, b3, Wc1, bc1, Wc2, bc2, Wc3, bc3):
    row = edge_index[0]
    col = edge_index[1]

    deg = jax.ops.segment_sum(jnp.ones((row.shape[0],), jnp.float32), col,
                              num_segments=_NPAD)[:, None]

    masks = jnp.zeros((4, _NPAD), jnp.float32)
    for i in range(4):
        masks = masks.at[i, com_xs[i]].set(1.0)
    masks = masks.T

    xp = jnp.pad(x, ((0, _NPAD - _N), (0, 0)))
    pe = jnp.pad(jnp.concatenate([pos_emb, lap_pe], axis=1),
                 ((0, _NPAD - _N), (0, 0)))

    x0, dinv = _tc_prologue(xp, pe, Wp, bp[None, :], deg)
    y1 = _tc_community(x0, masks, W1, b1, W2, b2, W3, b3, Wc1, dinv)
    acc1 = _edge_acc(y1, row, col)
    y2 = _tc_mid(acc1, y1, dinv, bc1[None, :], Wc2)
    acc2 = _edge_acc(y2, row, col)
    y3 = _tc_mid(acc2, y2, dinv, bc2[None, :], Wc3)
    acc3 = _edge_acc(y3, row, col)
    out = _tc_final(acc3, y3, dinv, bc3[None, :])
    return out[:_N]
